# baseline (device time: 22785 ns/iter reference)
import jax
import jax.numpy as jnp
from jax import lax
from jax.experimental import pallas as pl
from jax.experimental.pallas import tpu as pltpu


def kernel(dy, W):
    m, k = dy.shape
    d = W.shape[0]

    def body(dy_ref, w_ref, out_ref, send_buf, recv_buf, send_sem, recv_sem):
        my_x = lax.axis_index("x")
        my_y = lax.axis_index("y")
        my_z = lax.axis_index("z")

        partial = lax.dot_general(
            dy_ref[...].astype(jnp.bfloat16),
            w_ref[...].astype(jnp.bfloat16),
            dimension_numbers=(((1,), (1,)), ((), ())),
            preferred_element_type=jnp.float32,
        )
        send_buf[...] = partial.astype(jnp.bfloat16)

        rdma = pltpu.make_async_remote_copy(
            src_ref=send_buf,
            dst_ref=recv_buf,
            send_sem=send_sem,
            recv_sem=recv_sem,
            device_id=(my_x, 1 - my_y, my_z),
            device_id_type=pl.DeviceIdType.MESH,
        )
        rdma.start()
        rdma.wait()

        out_ref[...] = partial + recv_buf[...].astype(jnp.float32)

    return pl.pallas_call(
        body,
        out_shape=jax.ShapeDtypeStruct((m, d), jnp.float32),
        in_specs=[
            pl.BlockSpec(memory_space=pltpu.VMEM),
            pl.BlockSpec(memory_space=pltpu.VMEM),
        ],
        out_specs=pl.BlockSpec(memory_space=pltpu.VMEM),
        scratch_shapes=[
            pltpu.VMEM((m, d), jnp.bfloat16),
            pltpu.VMEM((m, d), jnp.bfloat16),
            pltpu.SemaphoreType.DMA,
            pltpu.SemaphoreType.DMA,
        ],
    )(dy, W)


# device time: 17710 ns/iter; 1.2866x vs baseline; 1.2866x over previous
import jax
import jax.numpy as jnp
from jax import lax
from jax.experimental import pallas as pl
from jax.experimental.pallas import tpu as pltpu


def kernel(dy, W):
    m, k = dy.shape
    d = W.shape[0]
    mh = m // 2

    def body(dy_ref, w_ref, out_ref,
             ysend, yrecv, xsend, xrecv,
             ysend_sem, yrecv_sem, xsend_sem, xrecv_sem):
        my_x = lax.axis_index("x")
        my_y = lax.axis_index("y")
        my_z = lax.axis_index("z")

        barrier = pltpu.get_barrier_semaphore()
        pl.semaphore_signal(
            barrier, inc=1, device_id=(my_x, 1 - my_y, my_z),
            device_id_type=pl.DeviceIdType.MESH)
        pl.semaphore_signal(
            barrier, inc=1, device_id=(1 - my_x, my_y, my_z),
            device_id_type=pl.DeviceIdType.MESH)

        part = lax.dot_general(
            dy_ref[pl.ds(my_x * mh, mh), :].astype(jnp.bfloat16),
            w_ref[...].astype(jnp.bfloat16),
            dimension_numbers=(((1,), (1,)), ((), ())),
            preferred_element_type=jnp.float32,
        )
        ysend[...] = part.astype(jnp.bfloat16)

        pl.semaphore_wait(barrier, 2)

        yrdma = pltpu.make_async_remote_copy(
            src_ref=ysend, dst_ref=yrecv,
            send_sem=ysend_sem, recv_sem=yrecv_sem,
            device_id=(my_x, 1 - my_y, my_z),
            device_id_type=pl.DeviceIdType.MESH,
        )
        yrdma.start()
        yrdma.wait()

        reduced = part + yrecv[...].astype(jnp.float32)
        out_ref[pl.ds(my_x * mh, mh), :] = reduced
        xsend[...] = reduced.astype(jnp.bfloat16)

        xrdma = pltpu.make_async_remote_copy(
            src_ref=xsend, dst_ref=xrecv,
            send_sem=xsend_sem, recv_sem=xrecv_sem,
            device_id=(1 - my_x, my_y, my_z),
            device_id_type=pl.DeviceIdType.MESH,
        )
        xrdma.start()
        xrdma.wait()

        out_ref[pl.ds((1 - my_x) * mh, mh), :] = xrecv[...].astype(jnp.float32)

    return pl.pallas_call(
        body,
        out_shape=jax.ShapeDtypeStruct((m, d), jnp.float32),
        in_specs=[
            pl.BlockSpec(memory_space=pltpu.VMEM),
            pl.BlockSpec(memory_space=pltpu.VMEM),
        ],
        out_specs=pl.BlockSpec(memory_space=pltpu.VMEM),
        scratch_shapes=[
            pltpu.VMEM((mh, d), jnp.bfloat16),
            pltpu.VMEM((mh, d), jnp.bfloat16),
            pltpu.VMEM((mh, d), jnp.bfloat16),
            pltpu.VMEM((mh, d), jnp.bfloat16),
            pltpu.SemaphoreType.DMA,
            pltpu.SemaphoreType.DMA,
            pltpu.SemaphoreType.DMA,
            pltpu.SemaphoreType.DMA,
        ],
        compiler_params=pltpu.CompilerParams(collective_id=0),
    )(dy, W)


# device time: 16349 ns/iter; 1.3937x vs baseline; 1.0832x over previous
import jax
import jax.numpy as jnp
from jax import lax
from jax.experimental import pallas as pl
from jax.experimental.pallas import tpu as pltpu

C = 4


def kernel(dy, W):
    m, k = dy.shape
    d = W.shape[0]
    mh = m // 2
    dc = d // C

    def body(dy_ref, w_ref, out_ref,
             dyv, wv, ysend, yrecv, xsend, xrecv, outv,
             load_sems, ysend_sems, yrecv_sems, xsend_sems, xrecv_sems,
             out_sems):
        my_x = lax.axis_index("x")
        my_y = lax.axis_index("y")
        my_z = lax.axis_index("z")

        barrier = pltpu.get_barrier_semaphore()
        pl.semaphore_signal(
            barrier, inc=1, device_id=(my_x, 1 - my_y, my_z),
            device_id_type=pl.DeviceIdType.MESH)
        pl.semaphore_signal(
            barrier, inc=1, device_id=(1 - my_x, my_y, my_z),
            device_id_type=pl.DeviceIdType.MESH)

        dy_dma = pltpu.make_async_copy(
            dy_ref.at[pl.ds(my_x * mh, mh), :], dyv, load_sems.at[0])
        dy_dma.start()
        w_dmas = []
        for c in range(C):
            wd = pltpu.make_async_copy(
                w_ref.at[pl.ds(c * dc, dc), :],
                wv.at[pl.ds(c * dc, dc), :],
                load_sems.at[1 + c])
            wd.start()
            w_dmas.append(wd)

        pl.semaphore_wait(barrier, 2)

        dy_dma.wait()
        dyb = dyv[...].astype(jnp.bfloat16)

        parts = []
        y_rdmas = []
        for c in range(C):
            w_dmas[c].wait()
            part_c = lax.dot_general(
                dyb,
                wv[pl.ds(c * dc, dc), :].astype(jnp.bfloat16),
                dimension_numbers=(((1,), (1,)), ((), ())),
                preferred_element_type=jnp.float32,
            )
            parts.append(part_c)
            ysend[c] = part_c.astype(jnp.bfloat16)
            yr = pltpu.make_async_remote_copy(
                src_ref=ysend.at[c], dst_ref=yrecv.at[c],
                send_sem=ysend_sems.at[c], recv_sem=yrecv_sems.at[c],
                device_id=(my_x, 1 - my_y, my_z),
                device_id_type=pl.DeviceIdType.MESH,
            )
            yr.start()
            y_rdmas.append(yr)

        x_rdmas = []
        for c in range(C):
            y_rdmas[c].wait()
            red_c = parts[c] + yrecv[c].astype(jnp.float32)
            outv[pl.ds(my_x * mh, mh), pl.ds(c * dc, dc)] = red_c
            xsend[c] = red_c.astype(jnp.bfloat16)
            xr = pltpu.make_async_remote_copy(
                src_ref=xsend.at[c], dst_ref=xrecv.at[c],
                send_sem=xsend_sems.at[c], recv_sem=xrecv_sems.at[c],
                device_id=(1 - my_x, my_y, my_z),
                device_id_type=pl.DeviceIdType.MESH,
            )
            xr.start()
            x_rdmas.append(xr)

        out_my = pltpu.make_async_copy(
            outv.at[pl.ds(my_x * mh, mh), :],
            out_ref.at[pl.ds(my_x * mh, mh), :],
            out_sems.at[0])
        out_my.start()

        for c in range(C):
            x_rdmas[c].wait()
            outv[pl.ds((1 - my_x) * mh, mh), pl.ds(c * dc, dc)] = (
                xrecv[c].astype(jnp.float32))

        out_other = pltpu.make_async_copy(
            outv.at[pl.ds((1 - my_x) * mh, mh), :],
            out_ref.at[pl.ds((1 - my_x) * mh, mh), :],
            out_sems.at[1])
        out_other.start()
        out_my.wait()
        out_other.wait()

    return pl.pallas_call(
        body,
        out_shape=jax.ShapeDtypeStruct((m, d), jnp.float32),
        in_specs=[
            pl.BlockSpec(memory_space=pl.ANY),
            pl.BlockSpec(memory_space=pl.ANY),
        ],
        out_specs=pl.BlockSpec(memory_space=pl.ANY),
        scratch_shapes=[
            pltpu.VMEM((mh, k), jnp.float32),
            pltpu.VMEM((d, k), jnp.float32),
            pltpu.VMEM((C, mh, dc), jnp.bfloat16),
            pltpu.VMEM((C, mh, dc), jnp.bfloat16),
            pltpu.VMEM((C, mh, dc), jnp.bfloat16),
            pltpu.VMEM((C, mh, dc), jnp.bfloat16),
            pltpu.VMEM((m, d), jnp.float32),
            pltpu.SemaphoreType.DMA((C + 1,)),
            pltpu.SemaphoreType.DMA((C,)),
            pltpu.SemaphoreType.DMA((C,)),
            pltpu.SemaphoreType.DMA((C,)),
            pltpu.SemaphoreType.DMA((C,)),
            pltpu.SemaphoreType.DMA((2,)),
        ],
        compiler_params=pltpu.CompilerParams(collective_id=0),
    )(dy, W)


# device time: 13283 ns/iter; 1.7154x vs baseline; 1.2308x over previous
import jax
import jax.numpy as jnp
from jax import lax
from jax.experimental import pallas as pl
from jax.experimental.pallas import tpu as pltpu

C = 4


def kernel(dy, W):
    dy = pltpu.with_memory_space_constraint(dy, pltpu.MemorySpace.HBM)
    W = pltpu.with_memory_space_constraint(W, pltpu.MemorySpace.HBM)
    m, k = dy.shape
    d = W.shape[0]
    mh = m // 2
    dc = d // C

    def body(dy_ref, w_ref, out_ref,
             dyv, wv, ysend, yrecv, xsend, xrecv, outv,
             load_sems, ysend_sems, yrecv_sems, xsend_sems, xrecv_sems,
             out_sems):
        my_x = lax.axis_index("x")
        my_y = lax.axis_index("y")
        my_z = lax.axis_index("z")

        barrier = pltpu.get_barrier_semaphore()
        pl.semaphore_signal(
            barrier, inc=1, device_id=(my_x, 1 - my_y, my_z),
            device_id_type=pl.DeviceIdType.MESH)
        pl.semaphore_signal(
            barrier, inc=1, device_id=(1 - my_x, my_y, my_z),
            device_id_type=pl.DeviceIdType.MESH)

        dy_dma = pltpu.make_async_copy(
            dy_ref.at[pl.ds(my_x * mh, mh), :], dyv, load_sems.at[0])
        dy_dma.start()
        w_dmas = []
        for c in range(C):
            wd = pltpu.make_async_copy(
                w_ref.at[pl.ds(c * dc, dc), :],
                wv.at[pl.ds(c * dc, dc), :],
                load_sems.at[1 + c])
            wd.start()
            w_dmas.append(wd)

        pl.semaphore_wait(barrier, 2)

        dy_dma.wait()
        dyb = dyv[...].astype(jnp.bfloat16)

        parts = []
        y_rdmas = []
        for c in range(C):
            w_dmas[c].wait()
            part_c = lax.dot_general(
                dyb,
                wv[pl.ds(c * dc, dc), :].astype(jnp.bfloat16),
                dimension_numbers=(((1,), (1,)), ((), ())),
                preferred_element_type=jnp.float32,
            )
            parts.append(part_c)
            ysend[c] = part_c.astype(jnp.bfloat16)
            yr = pltpu.make_async_remote_copy(
                src_ref=ysend.at[c], dst_ref=yrecv.at[c],
                send_sem=ysend_sems.at[c], recv_sem=yrecv_sems.at[c],
                device_id=(my_x, 1 - my_y, my_z),
                device_id_type=pl.DeviceIdType.MESH,
            )
            yr.start()
            y_rdmas.append(yr)

        x_rdmas = []
        for c in range(C):
            y_rdmas[c].wait()
            red_c = parts[c] + yrecv[c].astype(jnp.float32)
            outv[pl.ds(my_x * mh, mh), pl.ds(c * dc, dc)] = red_c
            xsend[c] = red_c.astype(jnp.bfloat16)
            xr = pltpu.make_async_remote_copy(
                src_ref=xsend.at[c], dst_ref=xrecv.at[c],
                send_sem=xsend_sems.at[c], recv_sem=xrecv_sems.at[c],
                device_id=(1 - my_x, my_y, my_z),
                device_id_type=pl.DeviceIdType.MESH,
            )
            xr.start()
            x_rdmas.append(xr)

        out_my = pltpu.make_async_copy(
            outv.at[pl.ds(my_x * mh, mh), :],
            out_ref.at[pl.ds(my_x * mh, mh), :],
            out_sems.at[0])
        out_my.start()

        for c in range(C):
            x_rdmas[c].wait()
            outv[pl.ds((1 - my_x) * mh, mh), pl.ds(c * dc, dc)] = (
                xrecv[c].astype(jnp.float32))

        out_other = pltpu.make_async_copy(
            outv.at[pl.ds((1 - my_x) * mh, mh), :],
            out_ref.at[pl.ds((1 - my_x) * mh, mh), :],
            out_sems.at[1])
        out_other.start()
        out_my.wait()
        out_other.wait()

    return pl.pallas_call(
        body,
        out_shape=jax.ShapeDtypeStruct((m, d), jnp.float32),
        in_specs=[
            pl.BlockSpec(memory_space=pltpu.MemorySpace.HBM),
            pl.BlockSpec(memory_space=pltpu.MemorySpace.HBM),
        ],
        out_specs=pl.BlockSpec(memory_space=pltpu.MemorySpace.HBM),
        scratch_shapes=[
            pltpu.VMEM((mh, k), jnp.float32),
            pltpu.VMEM((d, k), jnp.float32),
            pltpu.VMEM((C, mh, dc), jnp.bfloat16),
            pltpu.VMEM((C, mh, dc), jnp.bfloat16),
            pltpu.VMEM((C, mh, dc), jnp.bfloat16),
            pltpu.VMEM((C, mh, dc), jnp.bfloat16),
            pltpu.VMEM((m, d), jnp.float32),
            pltpu.SemaphoreType.DMA((C + 1,)),
            pltpu.SemaphoreType.DMA((C,)),
            pltpu.SemaphoreType.DMA((C,)),
            pltpu.SemaphoreType.DMA((C,)),
            pltpu.SemaphoreType.DMA((C,)),
            pltpu.SemaphoreType.DMA((2,)),
        ],
        compiler_params=pltpu.CompilerParams(collective_id=0),
    )(dy, W)


# device time: 13099 ns/iter; 1.7394x vs baseline; 1.0140x over previous
import jax
import jax.numpy as jnp
from jax import lax
from jax.experimental import pallas as pl
from jax.experimental.pallas import tpu as pltpu

C = 4


def kernel(dy, W):
    dy = pltpu.with_memory_space_constraint(dy, pltpu.MemorySpace.HBM)
    W = pltpu.with_memory_space_constraint(W, pltpu.MemorySpace.HBM)
    m, k = dy.shape
    d = W.shape[0]
    mh = m // 2
    dc = d // C

    def body(dy_ref, w_ref, out_ref,
             dyv, wv, ysend, yrecv, outv,
             load_sems, ysend_sems, yrecv_sems, xsend_sems, xrecv_sems,
             out_sems):
        my_x = lax.axis_index("x")
        my_y = lax.axis_index("y")
        my_z = lax.axis_index("z")

        barrier = pltpu.get_barrier_semaphore()
        pl.semaphore_signal(
            barrier, inc=1, device_id=(my_x, 1 - my_y, my_z),
            device_id_type=pl.DeviceIdType.MESH)
        pl.semaphore_signal(
            barrier, inc=1, device_id=(1 - my_x, my_y, my_z),
            device_id_type=pl.DeviceIdType.MESH)

        dy_dma = pltpu.make_async_copy(
            dy_ref.at[pl.ds(my_x * mh, mh), :], dyv, load_sems.at[0])
        dy_dma.start()
        w_dmas = []
        for c in range(C):
            wd = pltpu.make_async_copy(
                w_ref.at[pl.ds(c * dc, dc), :],
                wv.at[pl.ds(c * dc, dc), :],
                load_sems.at[1 + c])
            wd.start()
            w_dmas.append(wd)

        pl.semaphore_wait(barrier, 2)

        dy_dma.wait()
        dyb = dyv[...].astype(jnp.bfloat16)

        parts = []
        y_rdmas = []
        for c in range(C):
            w_dmas[c].wait()
            part_c = lax.dot_general(
                dyb,
                wv[pl.ds(c * dc, dc), :].astype(jnp.bfloat16),
                dimension_numbers=(((1,), (1,)), ((), ())),
                preferred_element_type=jnp.float32,
            )
            parts.append(part_c)
            ysend[c] = part_c.astype(jnp.bfloat16)
            yr = pltpu.make_async_remote_copy(
                src_ref=ysend.at[c], dst_ref=yrecv.at[c],
                send_sem=ysend_sems.at[c], recv_sem=yrecv_sems.at[c],
                device_id=(my_x, 1 - my_y, my_z),
                device_id_type=pl.DeviceIdType.MESH,
            )
            yr.start()
            y_rdmas.append(yr)

        x_rdmas = []
        for c in range(C):
            y_rdmas[c].wait()
            red_c = parts[c] + yrecv[c].astype(jnp.float32)
            outv[pl.ds(my_x * mh, mh), pl.ds(c * dc, dc)] = (
                red_c.astype(jnp.bfloat16))
            xr = pltpu.make_async_remote_copy(
                src_ref=outv.at[pl.ds(my_x * mh, mh), pl.ds(c * dc, dc)],
                dst_ref=outv.at[pl.ds(my_x * mh, mh), pl.ds(c * dc, dc)],
                send_sem=xsend_sems.at[c], recv_sem=xrecv_sems.at[c],
                device_id=(1 - my_x, my_y, my_z),
                device_id_type=pl.DeviceIdType.MESH,
            )
            xr.start()
            x_rdmas.append(xr)

        out_my = pltpu.make_async_copy(
            outv.at[pl.ds(my_x * mh, mh), :],
            out_ref.at[pl.ds(my_x * mh, mh), :],
            out_sems.at[0])
        out_my.start()

        for c in range(C):
            x_rdmas[c].wait()

        out_other = pltpu.make_async_copy(
            outv.at[pl.ds((1 - my_x) * mh, mh), :],
            out_ref.at[pl.ds((1 - my_x) * mh, mh), :],
            out_sems.at[1])
        out_other.start()
        out_my.wait()
        out_other.wait()

    return pl.pallas_call(
        body,
        out_shape=jax.ShapeDtypeStruct((m, d), jnp.bfloat16),
        in_specs=[
            pl.BlockSpec(memory_space=pltpu.MemorySpace.HBM),
            pl.BlockSpec(memory_space=pltpu.MemorySpace.HBM),
        ],
        out_specs=pl.BlockSpec(memory_space=pltpu.MemorySpace.HBM),
        scratch_shapes=[
            pltpu.VMEM((mh, k), jnp.float32),
            pltpu.VMEM((d, k), jnp.float32),
            pltpu.VMEM((C, mh, dc), jnp.bfloat16),
            pltpu.VMEM((C, mh, dc), jnp.bfloat16),
            pltpu.VMEM((m, d), jnp.bfloat16),
            pltpu.SemaphoreType.DMA((C + 1,)),
            pltpu.SemaphoreType.DMA((C,)),
            pltpu.SemaphoreType.DMA((C,)),
            pltpu.SemaphoreType.DMA((C,)),
            pltpu.SemaphoreType.DMA((C,)),
            pltpu.SemaphoreType.DMA((2,)),
        ],
        compiler_params=pltpu.CompilerParams(collective_id=0),
    )(dy, W)
